# baseline (speedup 1.0000x reference)
import functools

import jax
import jax.numpy as jnp
from jax import lax
from jax.experimental import pallas as pl
from jax.experimental.pallas import tpu as pltpu
from jax.experimental.pallas import tpu_sc as plsc

_N = 27554
_K = 9
_NC = 2
_NS = 16
_NW = _NC * _NS
_VPW = 864
_NPAD = _NW * _VPW
_G = _VPW // 16
_FLAT = 3 * _NPAD


def _dv_body(a_ref, b_ref, o_ref):
    o_ref[...] = a_ref[...] - b_ref[...]


def _dv_table(v1f, v2f):
    return pl.pallas_call(
        _dv_body,
        out_shape=jax.ShapeDtypeStruct((_FLAT // 128, 128), jnp.float32),
    )(v1f, v2f)


def _sc_body(dv_hbm, idx_hbm, w_hbm, lw_hbm, out_hbm, dv_v, idx_v, w_v, lw_v, tot_v):
    cid = lax.axis_index("c")
    sid = lax.axis_index("s")
    wid = sid * _NC + cid
    base = wid * _VPW
    pltpu.sync_copy(dv_hbm, dv_v)
    pltpu.sync_copy(idx_hbm.at[pl.ds(base * _K, _VPW * _K)], idx_v)
    pltpu.sync_copy(w_hbm.at[pl.ds(base, _VPW)], w_v)
    pltpu.sync_copy(lw_hbm.at[pl.ds(base, _VPW)], lw_v)
    lane = lax.iota(jnp.int32, 16)
    lane3 = lane * 3
    lane9 = lane * _K

    def body(g, tot):
        s0 = g * 16
        accx = jnp.zeros((16,), jnp.float32)
        accy = jnp.zeros((16,), jnp.float32)
        accz = jnp.zeros((16,), jnp.float32)
        for k in range(_K):
            i3 = plsc.load_gather(idx_v, [lane9 + (s0 * _K + k)]) * 3
            accx = accx + plsc.load_gather(dv_v, [i3])
            accy = accy + plsc.load_gather(dv_v, [i3 + 1])
            accz = accz + plsc.load_gather(dv_v, [i3 + 2])
        own = (base + s0) * 3 + lane3
        ox = plsc.load_gather(dv_v, [own])
        oy = plsc.load_gather(dv_v, [own + 1])
        oz = plsc.load_gather(dv_v, [own + 2])
        rcp = 1.0 / w_v[pl.ds(s0, 16)]
        tx = ox - accx * rcp
        ty = oy - accy * rcp
        tz = oz - accz * rcp
        return tot + lw_v[pl.ds(s0, 16)] * (tx * tx + ty * ty + tz * tz)

    tot = lax.fori_loop(0, _G, body, jnp.zeros((16,), jnp.float32))
    tot_v[...] = tot
    pltpu.sync_copy(tot_v, out_hbm.at[wid])


_sc_partials = functools.partial(
    pl.kernel,
    out_type=jax.ShapeDtypeStruct((_NW, 16), jnp.float32),
    mesh=plsc.VectorSubcoreMesh(core_axis_name="c", subcore_axis_name="s"),
    scratch_types=[
        pltpu.VMEM((_FLAT,), jnp.float32),
        pltpu.VMEM((_VPW * _K,), jnp.int32),
        pltpu.VMEM((_VPW,), jnp.float32),
        pltpu.VMEM((_VPW,), jnp.float32),
        pltpu.VMEM((16,), jnp.float32),
    ],
    compiler_params=pltpu.CompilerParams(needs_layout_passes=False),
)(_sc_body)


def kernel(v_1, v_2, adj_indices, adj_weights, laplace_w):
    v1f = jnp.pad(v_1.reshape(-1), (0, _FLAT - 3 * _N)).reshape(_FLAT // 128, 128)
    v2f = jnp.pad(v_2.reshape(-1), (0, _FLAT - 3 * _N)).reshape(_FLAT // 128, 128)
    dv = _dv_table(v1f, v2f).reshape(_FLAT)

    idx = adj_indices[:, :_K].astype(jnp.int32)
    idx = jnp.pad(idx, ((0, _NPAD - _N), (0, 0))).reshape(-1)

    w = jnp.pad(adj_weights.reshape(-1), (0, _NPAD - _N), constant_values=1.0)
    lw = jnp.pad(laplace_w.reshape(-1), (0, _NPAD - _N))

    partials = _sc_partials(dv, idx, w, lw)
    return jnp.sum(partials) / (_N * 3)


# speedup vs baseline: 1.2050x; 1.2050x over previous
import functools

import jax
import jax.numpy as jnp
from jax import lax
from jax.experimental import pallas as pl
from jax.experimental.pallas import tpu as pltpu
from jax.experimental.pallas import tpu_sc as plsc

_N = 27554
_K = 9
_NC = 2
_NS = 16
_NW = _NC * _NS
_VPW = 864
_NPAD = _NW * _VPW
_G = _VPW // 16
_RT = _NPAD // _NS
_GT = _RT // 16
_F_LAST = 3 * _N - 15 * 3 * _RT
_I_LAST = _K * _N - 31 * _K * _VPW
_VPW_LAST = _N - 31 * _VPW


def _sc_body(v1_hbm, v2_hbm, idx_hbm, w_hbm, lw_hbm, out_hbm,
             dv_v, v1c, v2c, idx_v, w_v, lw_v, tot_v, dv_sh):
    cid = lax.axis_index("c")
    sid = lax.axis_index("s")
    wid = sid * _NC + cid
    base = wid * _VPW
    lane = lax.iota(jnp.int32, 16)
    lane3 = lane * 3
    lane9 = lane * _K

    r0 = sid * _RT

    @pl.when(sid < _NS - 1)
    def _():
        pltpu.sync_copy(v1_hbm.at[pl.ds(r0 * 3, _RT * 3)], v1c)
        pltpu.sync_copy(v2_hbm.at[pl.ds(r0 * 3, _RT * 3)], v2c)

    @pl.when(sid == _NS - 1)
    def _():
        pltpu.sync_copy(v1_hbm.at[pl.ds(45 * _RT, _F_LAST)],
                        v1c.at[pl.ds(0, _F_LAST)])
        pltpu.sync_copy(v2_hbm.at[pl.ds(45 * _RT, _F_LAST)],
                        v2c.at[pl.ds(0, _F_LAST)])

    def stage(i, carry):
        f0 = lane3 + i * 48
        valid = (r0 + i * 16 + lane) < _N
        dx = plsc.load_gather(v1c, [f0]) - plsc.load_gather(v2c, [f0])
        dy = plsc.load_gather(v1c, [f0 + 1]) - plsc.load_gather(v2c, [f0 + 1])
        dz = plsc.load_gather(v1c, [f0 + 2]) - plsc.load_gather(v2c, [f0 + 2])
        zero = jnp.zeros((16,), jnp.float32)
        dv_v[pl.ds(r0 + i * 16, 16)] = jnp.where(valid, dx, zero)
        dv_v[pl.ds(_NPAD + r0 + i * 16, 16)] = jnp.where(valid, dy, zero)
        dv_v[pl.ds(2 * _NPAD + r0 + i * 16, 16)] = jnp.where(valid, dz, zero)
        return carry

    lax.fori_loop(0, _GT, stage, 0)
    pltpu.sync_copy(dv_v.at[pl.ds(r0, _RT)], dv_sh.at[pl.ds(r0, _RT)])
    pltpu.sync_copy(dv_v.at[pl.ds(_NPAD + r0, _RT)],
                    dv_sh.at[pl.ds(_NPAD + r0, _RT)])
    pltpu.sync_copy(dv_v.at[pl.ds(2 * _NPAD + r0, _RT)],
                    dv_sh.at[pl.ds(2 * _NPAD + r0, _RT)])
    plsc.subcore_barrier()
    pltpu.sync_copy(dv_sh, dv_v)

    @pl.when(wid < _NW - 1)
    def _():
        pltpu.sync_copy(idx_hbm.at[pl.ds(base * _K, _VPW * _K)], idx_v)
        pltpu.sync_copy(w_hbm.at[pl.ds(base, _VPW)], w_v)
        pltpu.sync_copy(lw_hbm.at[pl.ds(base, _VPW)], lw_v)

    @pl.when(wid == _NW - 1)
    def _():
        pltpu.sync_copy(idx_hbm.at[pl.ds(31 * _VPW * _K, _I_LAST)],
                        idx_v.at[pl.ds(0, _I_LAST)])
        pltpu.sync_copy(w_hbm.at[pl.ds(31 * _VPW, _VPW_LAST)],
                        w_v.at[pl.ds(0, _VPW_LAST)])
        pltpu.sync_copy(lw_hbm.at[pl.ds(31 * _VPW, _VPW_LAST)],
                        lw_v.at[pl.ds(0, _VPW_LAST)])

    def body(g, tot):
        s0 = g * 16
        gv = base + s0 + lane
        accx = jnp.zeros((16,), jnp.float32)
        accy = jnp.zeros((16,), jnp.float32)
        accz = jnp.zeros((16,), jnp.float32)
        for k in range(_K):
            iv = plsc.load_gather(idx_v, [lane9 + (s0 * _K + k)])
            iv = jnp.clip(iv, 0, _N)
            accx = accx + plsc.load_gather(dv_v, [iv])
            accy = accy + plsc.load_gather(dv_v, [iv + _NPAD])
            accz = accz + plsc.load_gather(dv_v, [iv + 2 * _NPAD])
        ox = plsc.load_gather(dv_v, [gv])
        oy = plsc.load_gather(dv_v, [gv + _NPAD])
        oz = plsc.load_gather(dv_v, [gv + 2 * _NPAD])
        rcp = 1.0 / w_v[pl.ds(s0, 16)]
        tx = ox - accx * rcp
        ty = oy - accy * rcp
        tz = oz - accz * rcp
        s = lw_v[pl.ds(s0, 16)] * (tx * tx + ty * ty + tz * tz)
        return tot + jnp.where(gv < _N, s, jnp.zeros((16,), jnp.float32))

    tot = lax.fori_loop(0, _G, body, jnp.zeros((16,), jnp.float32))
    tot_v[...] = tot
    pltpu.sync_copy(tot_v, out_hbm.at[wid])


_sc_loss = functools.partial(
    pl.kernel,
    out_type=jax.ShapeDtypeStruct((_NW, 16), jnp.float32),
    mesh=plsc.VectorSubcoreMesh(core_axis_name="c", subcore_axis_name="s"),
    scratch_types=[
        pltpu.VMEM((3 * _NPAD,), jnp.float32),
        pltpu.VMEM((3 * _RT,), jnp.float32),
        pltpu.VMEM((3 * _RT,), jnp.float32),
        pltpu.VMEM((_VPW * _K,), jnp.int32),
        pltpu.VMEM((_VPW,), jnp.float32),
        pltpu.VMEM((_VPW,), jnp.float32),
        pltpu.VMEM((16,), jnp.float32),
        pltpu.VMEM_SHARED((3 * _NPAD,), jnp.float32),
    ],
    compiler_params=pltpu.CompilerParams(needs_layout_passes=False,
                                         use_tc_tiling_on_sc=False),
)(_sc_body)


def kernel(v_1, v_2, adj_indices, adj_weights, laplace_w):
    idx = adj_indices[:, :_K].astype(jnp.int32).reshape(-1)
    partials = _sc_loss(v_1.reshape(-1), v_2.reshape(-1), idx,
                        adj_weights.reshape(-1), laplace_w.reshape(-1))
    return jnp.sum(partials) / (_N * 3)


# speedup vs baseline: 2.6672x; 2.2135x over previous
import functools

import jax
import jax.numpy as jnp
from jax import lax
from jax.experimental import pallas as pl
from jax.experimental.pallas import tpu as pltpu
from jax.experimental.pallas import tpu_sc as plsc

_N = 27554
_K = 9
_NC = 2
_NS = 16
_NW = _NC * _NS
_VPW = 864
_NPAD = _NW * _VPW
_G = _VPW // 16
_FLAT = 3 * _NPAD
_CH = _FLAT // _NS


def _sc_body(v1_hbm, v2_hbm, idx_hbm, w_hbm, lw_hbm, out_hbm,
             dv_v, vc1, vc2, idx_v, w_v, lw_v, tot_v, dv_sh):
    cid = lax.axis_index("c")
    sid = lax.axis_index("s")
    wid = sid * _NC + cid
    base = wid * _VPW
    lane = lax.iota(jnp.int32, 16)

    ch0 = sid * _CH
    pltpu.sync_copy(v1_hbm.at[pl.ds(ch0, _CH)], vc1)
    pltpu.sync_copy(v2_hbm.at[pl.ds(ch0, _CH)], vc2)

    def stage(i, carry):
        j = i * 16
        vc1[pl.ds(j, 16)] = vc1[pl.ds(j, 16)] - vc2[pl.ds(j, 16)]
        return carry

    lax.fori_loop(0, _CH // 16, stage, 0)
    pltpu.sync_copy(vc1, dv_sh.at[pl.ds(ch0, _CH)])
    plsc.subcore_barrier()
    pltpu.sync_copy(dv_sh, dv_v)

    for k in range(_K):
        pltpu.sync_copy(idx_hbm.at[pl.ds(k * _NPAD + base, _VPW)],
                        idx_v.at[pl.ds(k * _VPW, _VPW)])
    pltpu.sync_copy(w_hbm.at[pl.ds(base, _VPW)], w_v)
    pltpu.sync_copy(lw_hbm.at[pl.ds(base, _VPW)], lw_v)

    def body(g, tot):
        s0 = g * 16
        gv = base + s0 + lane
        accx = jnp.zeros((16,), jnp.float32)
        accy = jnp.zeros((16,), jnp.float32)
        accz = jnp.zeros((16,), jnp.float32)
        for k in range(_K):
            iv = idx_v[pl.ds(k * _VPW + s0, 16)]
            accx = accx + plsc.load_gather(dv_v, [iv])
            accy = accy + plsc.load_gather(dv_v, [iv + _NPAD])
            accz = accz + plsc.load_gather(dv_v, [iv + 2 * _NPAD])
        ox = plsc.load_gather(dv_v, [gv])
        oy = plsc.load_gather(dv_v, [gv + _NPAD])
        oz = plsc.load_gather(dv_v, [gv + 2 * _NPAD])
        rcp = 1.0 / w_v[pl.ds(s0, 16)]
        tx = ox - accx * rcp
        ty = oy - accy * rcp
        tz = oz - accz * rcp
        return tot + lw_v[pl.ds(s0, 16)] * (tx * tx + ty * ty + tz * tz)

    tot = lax.fori_loop(0, _G, body, jnp.zeros((16,), jnp.float32))
    tot_v[...] = tot
    pltpu.sync_copy(tot_v, out_hbm.at[wid])


_sc_loss = functools.partial(
    pl.kernel,
    out_type=jax.ShapeDtypeStruct((_NW, 16), jnp.float32),
    mesh=plsc.VectorSubcoreMesh(core_axis_name="c", subcore_axis_name="s"),
    scratch_types=[
        pltpu.VMEM((_FLAT,), jnp.float32),
        pltpu.VMEM((_CH,), jnp.float32),
        pltpu.VMEM((_CH,), jnp.float32),
        pltpu.VMEM((_VPW * _K,), jnp.int32),
        pltpu.VMEM((_VPW,), jnp.float32),
        pltpu.VMEM((_VPW,), jnp.float32),
        pltpu.VMEM((16,), jnp.float32),
        pltpu.VMEM_SHARED((_FLAT,), jnp.float32),
    ],
    compiler_params=pltpu.CompilerParams(needs_layout_passes=False,
                                         use_tc_tiling_on_sc=False),
)(_sc_body)


def kernel(v_1, v_2, adj_indices, adj_weights, laplace_w):
    pad = _NPAD - _N
    v1p = jnp.pad(v_1.T, ((0, 0), (0, pad))).reshape(-1)
    v2p = jnp.pad(v_2.T, ((0, 0), (0, pad))).reshape(-1)
    idxp = jnp.pad(adj_indices[:, :_K].astype(jnp.int32).T,
                   ((0, 0), (0, pad))).reshape(-1)
    wp = jnp.pad(adj_weights.reshape(-1), (0, pad), constant_values=1.0)
    lwp = jnp.pad(laplace_w.reshape(-1), (0, pad))
    partials = _sc_loss(v1p, v2p, idxp, wp, lwp)
    return jnp.sum(partials) / (_N * 3)
